# speedup vs baseline: 1.4218x; 1.4218x over previous
import functools

import jax
import jax.numpy as jnp
from jax import lax
from jax.experimental import pallas as pl
from jax.experimental.pallas import tpu as pltpu
from jax.experimental.pallas import tpu_sc as plsc

VOCAB = 52000
HIDDEN = 768
MAX_POS = 2048
EPS = 1e-12

_INFO = plsc.get_sparse_core_info()
_NC = _INFO.num_cores
_NS = _INFO.num_subcores
_NW = _NC * _NS

_TOKENS = 4 * 2048
_TOK_PER_W = _TOKENS // _NW
_CH = 64
_NCH = _TOK_PER_W // _CH


def _sc_gather(ids3d, token_table):
    mesh = plsc.VectorSubcoreMesh(core_axis_name="c", subcore_axis_name="s")

    @functools.partial(
        pl.kernel,
        mesh=mesh,
        out_type=jax.ShapeDtypeStruct((_TOKENS, HIDDEN), jnp.float32),
        scratch_types=[
            pltpu.VMEM((_NCH, _CH), jnp.int32),
            pltpu.VMEM((_CH, HIDDEN), jnp.float32),
            pltpu.VMEM((_CH, HIDDEN), jnp.float32),
            pltpu.SemaphoreType.DMA,
            pltpu.SemaphoreType.DMA,
        ],
    )
    def k(ids_ref, table_ref, out_ref, idx_v, buf0, buf1, sem0, sem1):
        wid = lax.axis_index("s") * _NC + lax.axis_index("c")
        base = wid * _TOK_PER_W
        pltpu.sync_copy(ids_ref.at[wid], idx_v)
        bufs = (buf0, buf1)
        sems = (sem0, sem1)
        cp = pltpu.async_copy(table_ref.at[idx_v.at[0]], bufs[0], sems[0])
        for c in range(_NCH):
            cp.wait()
            if c + 1 < _NCH:
                cp = pltpu.async_copy(
                    table_ref.at[idx_v.at[c + 1]], bufs[(c + 1) % 2],
                    sems[(c + 1) % 2])
            pltpu.sync_copy(bufs[c % 2],
                            out_ref.at[pl.ds(base + c * _CH, _CH)])

    return k(ids3d, token_table)


def _tc_ln_body(g_ref, p_ref, gamma_ref, beta_ref, o_ref):
    x = g_ref[...] + p_ref[...]
    mean = jnp.mean(x, axis=-1, keepdims=True)
    xc = x - mean
    var = jnp.mean(xc * xc, axis=-1, keepdims=True)
    o_ref[...] = (xc * lax.rsqrt(var + EPS)) * gamma_ref[...] + beta_ref[...]


def kernel(input_ids, token_table, pos_table, gamma, beta):
    B, S = input_ids.shape
    ids3d = input_ids.reshape(_NW, _NCH, _CH).astype(jnp.int32)
    gathered = _sc_gather(ids3d, token_table)

    BLK = 512
    grid = (_TOKENS // BLK,)
    blocks_per_seq = S // BLK
    out = pl.pallas_call(
        _tc_ln_body,
        grid=grid,
        in_specs=[
            pl.BlockSpec((BLK, HIDDEN), lambda i: (i, 0)),
            pl.BlockSpec((BLK, HIDDEN), lambda i: (i % blocks_per_seq, 0)),
            pl.BlockSpec((1, HIDDEN), lambda i: (0, 0)),
            pl.BlockSpec((1, HIDDEN), lambda i: (0, 0)),
        ],
        out_specs=pl.BlockSpec((BLK, HIDDEN), lambda i: (i, 0)),
        out_shape=jax.ShapeDtypeStruct((_TOKENS, HIDDEN), jnp.float32),
    )(gathered, pos_table, gamma.reshape(1, HIDDEN), beta.reshape(1, HIDDEN))
    return out.reshape(B, S, HIDDEN)


# speedup vs baseline: 1.4704x; 1.0341x over previous
import functools

import jax
import jax.numpy as jnp
from jax import lax
from jax.experimental import pallas as pl
from jax.experimental.pallas import tpu as pltpu
from jax.experimental.pallas import tpu_sc as plsc

VOCAB = 52000
HIDDEN = 768
MAX_POS = 2048
EPS = 1e-12

_INFO = plsc.get_sparse_core_info()
_NC = _INFO.num_cores
_NS = _INFO.num_subcores
_NW = _NC * _NS

_TOKENS = 4 * 2048
_TOK_PER_W = _TOKENS // _NW
_CH = 64
_NCH = _TOK_PER_W // _CH


def _sc_gather(ids3d, token_table):
    mesh = plsc.VectorSubcoreMesh(core_axis_name="c", subcore_axis_name="s")

    @functools.partial(
        pl.kernel,
        mesh=mesh,
        out_type=jax.ShapeDtypeStruct((_TOKENS, HIDDEN), jnp.float32),
        scratch_types=[
            pltpu.VMEM((_NCH, _CH), jnp.int32),
            pltpu.VMEM((_CH, HIDDEN), jnp.float32),
            pltpu.VMEM((_CH, HIDDEN), jnp.float32),
            pltpu.SemaphoreType.DMA,
            pltpu.SemaphoreType.DMA,
        ],
    )
    def k(ids_ref, table_ref, out_ref, idx_v, buf0, buf1, sem0, sem1):
        wid = lax.axis_index("s") * _NC + lax.axis_index("c")
        base = wid * _TOK_PER_W
        pltpu.sync_copy(ids_ref.at[wid], idx_v)
        bufs = (buf0, buf1)
        sems = (sem0, sem1)
        cp = pltpu.async_copy(table_ref.at[idx_v.at[0]], bufs[0], sems[0])
        for c in range(_NCH):
            cp.wait()
            if c + 1 < _NCH:
                cp = pltpu.async_copy(
                    table_ref.at[idx_v.at[c + 1]], bufs[(c + 1) % 2],
                    sems[(c + 1) % 2])
            pltpu.sync_copy(bufs[c % 2],
                            out_ref.at[pl.ds(base + c * _CH, _CH)])

    return k(ids3d, token_table)


def _tc_ln_body(g_ref, p_ref, gamma_ref, beta_ref, o_ref):
    x = g_ref[...] + p_ref[...]
    mean = jnp.mean(x, axis=-1, keepdims=True)
    xc = x - mean
    var = jnp.mean(xc * xc, axis=-1, keepdims=True)
    o_ref[...] = (xc * lax.rsqrt(var + EPS)) * gamma_ref[...] + beta_ref[...]


def kernel(input_ids, token_table, pos_table, gamma, beta):
    B, S = input_ids.shape
    ids3d = input_ids.reshape(_NW, _NCH, _CH).astype(jnp.int32)
    gathered = _sc_gather(ids3d, token_table)

    BLK = 512
    blocks_per_seq = S // BLK
    out = pl.pallas_call(
        _tc_ln_body,
        grid=(blocks_per_seq, B),
        in_specs=[
            pl.BlockSpec((BLK, HIDDEN), lambda i, j: (j * blocks_per_seq + i, 0)),
            pl.BlockSpec((BLK, HIDDEN), lambda i, j: (i, 0)),
            pl.BlockSpec((1, HIDDEN), lambda i, j: (0, 0)),
            pl.BlockSpec((1, HIDDEN), lambda i, j: (0, 0)),
        ],
        out_specs=pl.BlockSpec((BLK, HIDDEN), lambda i, j: (j * blocks_per_seq + i, 0)),
        out_shape=jax.ShapeDtypeStruct((_TOKENS, HIDDEN), jnp.float32),
    )(gathered, pos_table, gamma.reshape(1, HIDDEN), beta.reshape(1, HIDDEN))
    return out.reshape(B, S, HIDDEN)


# speedup vs baseline: 1.5525x; 1.0558x over previous
import functools

import jax
import jax.numpy as jnp
from jax import lax
from jax.experimental import pallas as pl
from jax.experimental.pallas import tpu as pltpu
from jax.experimental.pallas import tpu_sc as plsc

VOCAB = 52000
HIDDEN = 768
MAX_POS = 2048
EPS = 1e-12

_INFO = plsc.get_sparse_core_info()
_NC = _INFO.num_cores
_NS = _INFO.num_subcores
_NW = _NC * _NS

_TOKENS = 4 * 2048
_TOK_PER_W = _TOKENS // _NW
_CH = 64
_NCH = _TOK_PER_W // _CH


def _sc_gather(ids3d, token_table):
    mesh = plsc.VectorSubcoreMesh(core_axis_name="c", subcore_axis_name="s")

    @functools.partial(
        pl.kernel,
        mesh=mesh,
        out_type=jax.ShapeDtypeStruct((_TOKENS, HIDDEN), jnp.float32),
        scratch_types=[
            pltpu.VMEM((_NCH, _CH), jnp.int32),
            pltpu.VMEM((_CH, HIDDEN), jnp.float32),
            pltpu.VMEM((_CH, HIDDEN), jnp.float32),
            pltpu.SemaphoreType.DMA,
            pltpu.SemaphoreType.DMA,
        ],
    )
    def k(ids_ref, table_ref, out_ref, idx_v, buf0, buf1, sem0, sem1):
        wid = lax.axis_index("s") * _NC + lax.axis_index("c")
        base = wid * _TOK_PER_W
        pltpu.sync_copy(ids_ref.at[wid], idx_v)
        bufs = (buf0, buf1)
        sems = (sem0, sem1)
        cp = pltpu.async_copy(table_ref.at[idx_v.at[0]], bufs[0], sems[0])
        for c in range(_NCH):
            cp.wait()
            if c + 1 < _NCH:
                cp = pltpu.async_copy(
                    table_ref.at[idx_v.at[c + 1]], bufs[(c + 1) % 2],
                    sems[(c + 1) % 2])
            pltpu.sync_copy(bufs[c % 2],
                            out_ref.at[pl.ds(base + c * _CH, _CH)])

    return k(ids3d, token_table)


def _tc_ln_body(g_ref, p_ref, gamma_ref, beta_ref, o_ref):
    x = g_ref[...] + p_ref[...]
    mean = jnp.mean(x, axis=-1, keepdims=True)
    xc = x - mean
    var = jnp.mean(xc * xc, axis=-1, keepdims=True)
    o_ref[...] = (xc * lax.rsqrt(var + EPS)) * gamma_ref[...] + beta_ref[...]


def kernel(input_ids, token_table, pos_table, gamma, beta):
    B, S = input_ids.shape
    ids3d = input_ids.reshape(_NW, _NCH, _CH).astype(jnp.int32)
    gathered = _sc_gather(ids3d, token_table)

    BLK = 1024
    blocks_per_seq = S // BLK
    out = pl.pallas_call(
        _tc_ln_body,
        grid=(blocks_per_seq, B),
        in_specs=[
            pl.BlockSpec((BLK, HIDDEN), lambda i, j: (j * blocks_per_seq + i, 0)),
            pl.BlockSpec((BLK, HIDDEN), lambda i, j: (i, 0)),
            pl.BlockSpec((1, HIDDEN), lambda i, j: (0, 0)),
            pl.BlockSpec((1, HIDDEN), lambda i, j: (0, 0)),
        ],
        out_specs=pl.BlockSpec((BLK, HIDDEN), lambda i, j: (j * blocks_per_seq + i, 0)),
        out_shape=jax.ShapeDtypeStruct((_TOKENS, HIDDEN), jnp.float32),
    )(gathered, pos_table, gamma.reshape(1, HIDDEN), beta.reshape(1, HIDDEN))
    return out.reshape(B, S, HIDDEN)


# speedup vs baseline: 1.5815x; 1.0187x over previous
import functools

import jax
import jax.numpy as jnp
from jax import lax
from jax.experimental import pallas as pl
from jax.experimental.pallas import tpu as pltpu
from jax.experimental.pallas import tpu_sc as plsc

VOCAB = 52000
HIDDEN = 768
MAX_POS = 2048
EPS = 1e-12

_INFO = plsc.get_sparse_core_info()
_NC = _INFO.num_cores
_NS = _INFO.num_subcores
_NW = _NC * _NS

_TOKENS = 4 * 2048
_TOK_PER_W = _TOKENS // _NW
_CH = 64
_NCH = _TOK_PER_W // _CH


def _sc_gather(ids3d, token_table):
    mesh = plsc.VectorSubcoreMesh(core_axis_name="c", subcore_axis_name="s")

    @functools.partial(
        pl.kernel,
        mesh=mesh,
        out_type=jax.ShapeDtypeStruct((_TOKENS, HIDDEN), jnp.float32),
        scratch_types=[
            pltpu.VMEM((_NCH, _CH), jnp.int32),
            pltpu.VMEM((_CH, HIDDEN), jnp.float32),
            pltpu.VMEM((_CH, HIDDEN), jnp.float32),
            pltpu.SemaphoreType.DMA,
            pltpu.SemaphoreType.DMA,
        ],
    )
    def k(ids_ref, table_ref, out_ref, idx_v, buf0, buf1, sem0, sem1):
        wid = lax.axis_index("s") * _NC + lax.axis_index("c")
        base = wid * _TOK_PER_W
        pltpu.sync_copy(ids_ref.at[wid], idx_v)
        bufs = (buf0, buf1)
        sems = (sem0, sem1)
        cp = pltpu.async_copy(table_ref.at[idx_v.at[0]], bufs[0], sems[0])
        for c in range(_NCH):
            cp.wait()
            if c + 1 < _NCH:
                cp = pltpu.async_copy(
                    table_ref.at[idx_v.at[c + 1]], bufs[(c + 1) % 2],
                    sems[(c + 1) % 2])
            pltpu.sync_copy(bufs[c % 2],
                            out_ref.at[pl.ds(base + c * _CH, _CH)])

    return k(ids3d, token_table)


def _tc_ln_body(g_ref, p_ref, gamma_ref, beta_ref, o_ref):
    x = g_ref[...] + p_ref[...]
    mean = jnp.mean(x, axis=-1, keepdims=True)
    xc = x - mean
    var = jnp.mean(xc * xc, axis=-1, keepdims=True)
    o_ref[...] = (xc * lax.rsqrt(var + EPS)) * gamma_ref[...] + beta_ref[...]


def kernel(input_ids, token_table, pos_table, gamma, beta):
    B, S = input_ids.shape
    ids3d = input_ids.reshape(_NW, _NCH, _CH).astype(jnp.int32)
    gathered = _sc_gather(ids3d, token_table)

    BLK = 2048
    blocks_per_seq = S // BLK
    out = pl.pallas_call(
        _tc_ln_body,
        grid=(blocks_per_seq, B),
        in_specs=[
            pl.BlockSpec((BLK, HIDDEN), lambda i, j: (j * blocks_per_seq + i, 0)),
            pl.BlockSpec((BLK, HIDDEN), lambda i, j: (i, 0)),
            pl.BlockSpec((1, HIDDEN), lambda i, j: (0, 0)),
            pl.BlockSpec((1, HIDDEN), lambda i, j: (0, 0)),
        ],
        out_specs=pl.BlockSpec((BLK, HIDDEN), lambda i, j: (j * blocks_per_seq + i, 0)),
        out_shape=jax.ShapeDtypeStruct((_TOKENS, HIDDEN), jnp.float32),
    )(gathered, pos_table, gamma.reshape(1, HIDDEN), beta.reshape(1, HIDDEN))
    return out.reshape(B, S, HIDDEN)
